# baseline (speedup 1.0000x reference)
import functools
import math

import jax
import jax.numpy as jnp
from jax import lax
from jax.experimental import pallas as pl
from jax.experimental.pallas import tpu as pltpu
from jax.experimental.pallas import tpu_sc as plsc

_NUM_BINS = 9
_CELL = 8
_PI = math.pi

_B, _C, _H, _W = 8, 3, 512, 512
_NHC, _NWC = _H // _CELL, _W // _CELL
_NCORES, _NSUB = 2, 16
_NW = _NCORES * _NSUB

_SPLIT = 1
_NIMG = _B // _SPLIT

_WPI = _NW // _NIMG
_QROWS = _H // _WPI
_QCROWS = _QROWS // _CELL
_PLANE = _QCROWS * _NWC
_HIST = _NUM_BINS * _PLANE
_BANKS = _CELL
_BSTRIDE = _HIST + 1
_HWORDS = (_BANKS * _BSTRIDE + 255) // 256 * 256

_ATAN_C = (
    0.99997726,
    -0.33262347,
    0.19354346,
    -0.11643287,
    0.05265332,
    -0.01172120,
)


def _atan01(a):
    z = a * a
    p = jnp.float32(_ATAN_C[5])
    for c in _ATAN_C[4::-1]:
        p = p * z + jnp.float32(c)
    return a * p


def _stage1_body(x_ref, cell_ref, wi_ref, wv_ref):
    x = x_ref[0]
    C, H, W = x.shape

    zc = jnp.zeros((C, H, 1), jnp.float32)
    gx = jnp.concatenate([zc, x[:, :, 2:] - x[:, :, :-2], zc], axis=2)
    zr = jnp.zeros((C, 1, W), jnp.float32)
    gy = jnp.concatenate([zr, x[:, 2:, :] - x[:, :-2, :], zr], axis=1)

    m = gx * gx + gy * gy + 1e-12
    m0, m1, m2 = m[0], m[1], m[2]
    c0 = (m0 >= m1) & (m0 >= m2)
    c1 = jnp.logical_and(~c0, m1 >= m2)
    gxs = jnp.where(c0, gx[0], jnp.where(c1, gx[1], gx[2]))
    gys = jnp.where(c0, gy[0], jnp.where(c1, gy[1], gy[2]))
    mag = jnp.sqrt(jnp.where(c0, m0, jnp.where(c1, m1, m2)))

    ax = jnp.abs(gxs)
    ay = jnp.abs(gys)
    r = _atan01(jnp.minimum(ax, ay) / jnp.maximum(jnp.maximum(ax, ay), 1e-30))
    phi = jnp.where(ay > ax, jnp.float32(0.5 * _PI) - r, r)
    neg = (lax.bitcast_convert_type(gxs, jnp.int32)
           ^ lax.bitcast_convert_type(gys, jnp.int32)) < 0
    theta = jnp.where(neg, jnp.float32(_PI) - phi, phi)

    b = theta * jnp.float32(_NUM_BINS / _PI) - 0.5
    b0 = jnp.floor(b)
    w1 = b - b0
    v1 = mag * w1
    v0 = mag - v1
    k0 = jnp.where(b0 < 0, jnp.float32(_NUM_BINS - 1), b0).astype(jnp.int32)
    k1 = jnp.where(k0 >= _NUM_BINS - 1, 0, k0 + 1)

    cell = cell_ref[...]
    idx0 = k0 * _PLANE + cell
    idx1 = k1 * _PLANE + cell
    wi_ref[0] = idx0 | (idx1 << 16)

    b0i = lax.bitcast_convert_type(v0, jnp.int32)
    b1i = lax.bitcast_convert_type(v1, jnp.int32)
    wv_ref[0] = (b0i & jnp.int32(-65536)) | lax.shift_right_logical(
        b1i, 16).astype(jnp.int32)


def _sc_body(wi_hbm, wv_hbm, out_hbm, buf_i, buf_v, hist, h3, sem0, sem1):
    cid = lax.axis_index("c")
    sid = lax.axis_index("s")
    wid = sid * _NCORES + cid
    b = lax.shift_right_logical(wid, _WPI.bit_length() - 1)
    q = wid & (_WPI - 1)

    def _start(c, slot, sem):
        r0 = q * _QROWS + c * 8
        pltpu.async_copy(wi_hbm.at[b, pl.ds(r0, 8), :], buf_i.at[slot], sem)
        pltpu.async_copy(wv_hbm.at[b, pl.ds(r0, 8), :], buf_v.at[slot], sem)

    def _drain(slot, sem):
        pltpu.make_async_copy(
            wi_hbm.at[b, pl.ds(0, 8), :], buf_i.at[slot], sem).wait()
        pltpu.make_async_copy(
            wv_hbm.at[b, pl.ds(0, 8), :], buf_v.at[slot], sem).wait()

    _start(0, 0, sem0)

    zeros16 = jnp.zeros((16,), jnp.float32)

    @plsc.parallel_loop(0, _HWORDS // 256)
    def _zero(i):
        for j in range(16):
            hist[pl.ds(i * 256 + j * 16, 16)] = zeros16

    lane = lax.broadcasted_iota(jnp.int32, (16,), 0)
    loff = (lane & (_BANKS - 1)) * _BSTRIDE
    lo16 = jnp.int32(0xFFFF)

    def _consume(slot):
        @plsc.parallel_loop(0, _W // 16, unroll=4)
        def _vec(i):
            for rr in range(8):
                wiv = buf_i[slot, rr, pl.ds(i * 16, 16)]
                wvv = buf_v[slot, rr, pl.ds(i * 16, 16)]
                i0 = (wiv & lo16) + loff
                i1 = lax.shift_right_logical(wiv, 16).astype(jnp.int32) + loff
                v0 = lax.bitcast_convert_type(
                    wvv & jnp.int32(-65536), jnp.float32)
                v1 = lax.bitcast_convert_type(
                    lax.shift_left(wvv, 16), jnp.float32)
                plsc.addupdate_scatter(hist, [i0], v0)
                plsc.addupdate_scatter(hist, [i1], v1)

    def _pair(t, carry):
        _start(2 * t + 1, 1, sem1)
        _drain(0, sem0)
        _consume(0)

        @pl.when(t < _QROWS // 16 - 1)
        def _():
            _start(2 * t + 2, 0, sem0)

        _drain(1, sem1)
        _consume(1)
        return carry

    lax.fori_loop(0, _QROWS // 16, _pair, 0)

    @plsc.parallel_loop(0, _HIST // 32, unroll=2)
    def _reduce(v):
        for j in range(2):
            base = v * 32 + j * 16
            acc = hist[pl.ds(base, 16)]
            for l8 in range(1, _BANKS):
                acc = acc + hist[pl.ds(l8 * _BSTRIDE + base, 16)]
            hist[pl.ds(base, 16)] = acc

    for j in range(_PLANE // 16):
        lr, c4 = j // 4, j % 4
        base = lr * _NWC + c4 * 16
        s = jnp.full((16,), 1e-6, jnp.float32)
        vals = []
        for k in range(_NUM_BINS):
            hv = hist[pl.ds(k * _PLANE + base, 16)]
            vals.append(hv)
            s = s + hv * hv
        ybits = jnp.int32(0x5F3759DF) - lax.shift_right_logical(
            lax.bitcast_convert_type(s, jnp.int32), 1).astype(jnp.int32)
        y = lax.bitcast_convert_type(ybits, jnp.float32)
        for _ in range(3):
            y = y * (1.5 - 0.5 * s * y * y)
        for k in range(_NUM_BINS):
            h3[k, lr, pl.ds(c4 * 16, 16)] = vals[k] * y

    pltpu.sync_copy(h3, out_hbm.at[b, :, pl.ds(q * _QCROWS, _QCROWS), :])


_sc_stage = functools.partial(
    pl.kernel,
    mesh=plsc.VectorSubcoreMesh(core_axis_name="c", subcore_axis_name="s"),
    out_type=jax.ShapeDtypeStruct((_NIMG, _NUM_BINS, _NHC, _NWC), jnp.float32),
    compiler_params=pltpu.CompilerParams(needs_layout_passes=False),
    scratch_types=[
        pltpu.VMEM((2, 8, _W), jnp.int32),
        pltpu.VMEM((2, 8, _W), jnp.int32),
        pltpu.VMEM((_HWORDS,), jnp.float32),
        pltpu.VMEM((_NUM_BINS, _QCROWS, _NWC), jnp.float32),
        pltpu.SemaphoreType.DMA,
        pltpu.SemaphoreType.DMA,
    ],
)(_sc_body)


def _stage1(imgs):
    n, C, H, W = imgs.shape
    hh = jnp.arange(H, dtype=jnp.int32)[:, None]
    ww = jnp.arange(W, dtype=jnp.int32)[None, :]
    cell = ((hh & (_QROWS - 1)) >> 3) * _NWC + (ww >> 3)
    return pl.pallas_call(
        _stage1_body,
        grid=(n,),
        in_specs=[
            pl.BlockSpec((1, C, H, W), lambda b: (b, 0, 0, 0)),
            pl.BlockSpec((H, W), lambda b: (0, 0)),
        ],
        out_specs=[
            pl.BlockSpec((1, H, W), lambda b: (b, 0, 0)),
            pl.BlockSpec((1, H, W), lambda b: (b, 0, 0)),
        ],
        out_shape=[
            jax.ShapeDtypeStruct((n, H, W), jnp.int32),
            jax.ShapeDtypeStruct((n, H, W), jnp.int32),
        ],
    )(imgs, cell)


def kernel(img):
    outs = []
    for s in range(_SPLIT):
        wi, wv = _stage1(lax.slice_in_dim(img, s * _NIMG, (s + 1) * _NIMG))
        outs.append(_sc_stage(wi, wv))
    return jnp.concatenate(outs, axis=0)


# speedup vs baseline: 1.0092x; 1.0092x over previous
import functools
import math

import jax
import jax.numpy as jnp
from jax import lax
from jax.experimental import pallas as pl
from jax.experimental.pallas import tpu as pltpu
from jax.experimental.pallas import tpu_sc as plsc

_NUM_BINS = 9
_CELL = 8
_PI = math.pi

_B, _C, _H, _W = 8, 3, 512, 512
_NHC, _NWC = _H // _CELL, _W // _CELL
_NCORES, _NSUB = 2, 16
_NW = _NCORES * _NSUB

_SPLIT = 1
_NIMG = _B // _SPLIT

_WPI = _NW // _NIMG
_QROWS = _H // _WPI
_QCROWS = _QROWS // _CELL
_PLANE = _QCROWS * _NWC
_HIST = _NUM_BINS * _PLANE
_BANKS = _CELL
_BSTRIDE = _HIST + 1
_HWORDS = (_BANKS * _BSTRIDE + 255) // 256 * 256

_ATAN_C = (
    0.99997726,
    -0.33262347,
    0.19354346,
    -0.11643287,
    0.05265332,
    -0.01172120,
)


def _atan01(a):
    z = a * a
    p = jnp.float32(_ATAN_C[5])
    for c in _ATAN_C[4::-1]:
        p = p * z + jnp.float32(c)
    return a * p


def _stage1_body(x_ref, cell_ref, wi_ref, wv_ref):
    x = x_ref[0]
    C, H, W = x.shape

    zc = jnp.zeros((C, H, 1), jnp.float32)
    gx = jnp.concatenate([zc, x[:, :, 2:] - x[:, :, :-2], zc], axis=2)
    zr = jnp.zeros((C, 1, W), jnp.float32)
    gy = jnp.concatenate([zr, x[:, 2:, :] - x[:, :-2, :], zr], axis=1)

    m = gx * gx + gy * gy + 1e-12
    m0, m1, m2 = m[0], m[1], m[2]
    c0 = (m0 >= m1) & (m0 >= m2)
    c1 = jnp.logical_and(~c0, m1 >= m2)
    gxs = jnp.where(c0, gx[0], jnp.where(c1, gx[1], gx[2]))
    gys = jnp.where(c0, gy[0], jnp.where(c1, gy[1], gy[2]))
    mag = jnp.sqrt(jnp.where(c0, m0, jnp.where(c1, m1, m2)))

    ax = jnp.abs(gxs)
    ay = jnp.abs(gys)
    r = _atan01(jnp.minimum(ax, ay) / jnp.maximum(jnp.maximum(ax, ay), 1e-30))
    phi = jnp.where(ay > ax, jnp.float32(0.5 * _PI) - r, r)
    neg = (lax.bitcast_convert_type(gxs, jnp.int32)
           ^ lax.bitcast_convert_type(gys, jnp.int32)) < 0
    theta = jnp.where(neg, jnp.float32(_PI) - phi, phi)

    b = theta * jnp.float32(_NUM_BINS / _PI) - 0.5
    b0 = jnp.floor(b)
    w1 = b - b0
    v1 = mag * w1
    v0 = mag - v1
    k0 = jnp.where(b0 < 0, jnp.float32(_NUM_BINS - 1), b0).astype(jnp.int32)
    k1 = jnp.where(k0 >= _NUM_BINS - 1, 0, k0 + 1)

    cell = cell_ref[...]
    idx0 = k0 * _PLANE + cell
    idx1 = k1 * _PLANE + cell
    wi_ref[0] = idx0 | (idx1 << 16)

    b0i = lax.bitcast_convert_type(v0, jnp.int32)
    b1i = lax.bitcast_convert_type(v1, jnp.int32)
    wv_ref[0] = (b0i & jnp.int32(-65536)) | lax.shift_right_logical(
        b1i, 16).astype(jnp.int32)


def _sc_body(wi_hbm, wv_hbm, out_hbm, buf_i, buf_v, hist, h3, sem0, sem1):
    cid = lax.axis_index("c")
    sid = lax.axis_index("s")
    wid = sid * _NCORES + cid
    b = lax.shift_right_logical(wid, _WPI.bit_length() - 1)
    q = wid & (_WPI - 1)

    def _start(c, slot, sem):
        r0 = q * _QROWS + c * 16
        pltpu.async_copy(wi_hbm.at[b, pl.ds(r0, 16), :], buf_i.at[slot], sem)
        pltpu.async_copy(wv_hbm.at[b, pl.ds(r0, 16), :], buf_v.at[slot], sem)

    def _drain(slot, sem):
        pltpu.make_async_copy(
            wi_hbm.at[b, pl.ds(0, 16), :], buf_i.at[slot], sem).wait()
        pltpu.make_async_copy(
            wv_hbm.at[b, pl.ds(0, 16), :], buf_v.at[slot], sem).wait()

    _start(0, 0, sem0)

    zeros16 = jnp.zeros((16,), jnp.float32)

    @plsc.parallel_loop(0, _HWORDS // 256)
    def _zero(i):
        for j in range(16):
            hist[pl.ds(i * 256 + j * 16, 16)] = zeros16

    lane = lax.broadcasted_iota(jnp.int32, (16,), 0)
    loff = (lane & (_BANKS - 1)) * _BSTRIDE
    lo16 = jnp.int32(0xFFFF)

    def _consume(slot):
        @plsc.parallel_loop(0, _W // 16, unroll=2)
        def _vec(i):
            for rr in range(16):
                wiv = buf_i[slot, rr, pl.ds(i * 16, 16)]
                wvv = buf_v[slot, rr, pl.ds(i * 16, 16)]
                i0 = (wiv & lo16) + loff
                i1 = lax.shift_right_logical(wiv, 16).astype(jnp.int32) + loff
                v0 = lax.bitcast_convert_type(
                    wvv & jnp.int32(-65536), jnp.float32)
                v1 = lax.bitcast_convert_type(
                    lax.shift_left(wvv, 16), jnp.float32)
                plsc.addupdate_scatter(hist, [i0], v0)
                plsc.addupdate_scatter(hist, [i1], v1)

    def _pair(t, carry):
        _start(2 * t + 1, 1, sem1)
        _drain(0, sem0)
        _consume(0)

        @pl.when(t < _QROWS // 32 - 1)
        def _():
            _start(2 * t + 2, 0, sem0)

        _drain(1, sem1)
        _consume(1)
        return carry

    lax.fori_loop(0, _QROWS // 32, _pair, 0)

    @plsc.parallel_loop(0, _HIST // 32, unroll=2)
    def _reduce(v):
        for j in range(2):
            base = v * 32 + j * 16
            acc = hist[pl.ds(base, 16)]
            for l8 in range(1, _BANKS):
                acc = acc + hist[pl.ds(l8 * _BSTRIDE + base, 16)]
            hist[pl.ds(base, 16)] = acc

    for j in range(_PLANE // 16):
        lr, c4 = j // 4, j % 4
        base = lr * _NWC + c4 * 16
        s = jnp.full((16,), 1e-6, jnp.float32)
        vals = []
        for k in range(_NUM_BINS):
            hv = hist[pl.ds(k * _PLANE + base, 16)]
            vals.append(hv)
            s = s + hv * hv
        ybits = jnp.int32(0x5F3759DF) - lax.shift_right_logical(
            lax.bitcast_convert_type(s, jnp.int32), 1).astype(jnp.int32)
        y = lax.bitcast_convert_type(ybits, jnp.float32)
        for _ in range(3):
            y = y * (1.5 - 0.5 * s * y * y)
        for k in range(_NUM_BINS):
            h3[k, lr, pl.ds(c4 * 16, 16)] = vals[k] * y

    pltpu.sync_copy(h3, out_hbm.at[b, :, pl.ds(q * _QCROWS, _QCROWS), :])


_sc_stage = functools.partial(
    pl.kernel,
    mesh=plsc.VectorSubcoreMesh(core_axis_name="c", subcore_axis_name="s"),
    out_type=jax.ShapeDtypeStruct((_NIMG, _NUM_BINS, _NHC, _NWC), jnp.float32),
    compiler_params=pltpu.CompilerParams(needs_layout_passes=False),
    scratch_types=[
        pltpu.VMEM((2, 16, _W), jnp.int32),
        pltpu.VMEM((2, 16, _W), jnp.int32),
        pltpu.VMEM((_HWORDS,), jnp.float32),
        pltpu.VMEM((_NUM_BINS, _QCROWS, _NWC), jnp.float32),
        pltpu.SemaphoreType.DMA,
        pltpu.SemaphoreType.DMA,
    ],
)(_sc_body)


def _stage1(imgs):
    n, C, H, W = imgs.shape
    hh = jnp.arange(H, dtype=jnp.int32)[:, None]
    ww = jnp.arange(W, dtype=jnp.int32)[None, :]
    cell = ((hh & (_QROWS - 1)) >> 3) * _NWC + (ww >> 3)
    return pl.pallas_call(
        _stage1_body,
        grid=(n,),
        in_specs=[
            pl.BlockSpec((1, C, H, W), lambda b: (b, 0, 0, 0)),
            pl.BlockSpec((H, W), lambda b: (0, 0)),
        ],
        out_specs=[
            pl.BlockSpec((1, H, W), lambda b: (b, 0, 0)),
            pl.BlockSpec((1, H, W), lambda b: (b, 0, 0)),
        ],
        out_shape=[
            jax.ShapeDtypeStruct((n, H, W), jnp.int32),
            jax.ShapeDtypeStruct((n, H, W), jnp.int32),
        ],
    )(imgs, cell)


def kernel(img):
    outs = []
    for s in range(_SPLIT):
        wi, wv = _stage1(lax.slice_in_dim(img, s * _NIMG, (s + 1) * _NIMG))
        outs.append(_sc_stage(wi, wv))
    return jnp.concatenate(outs, axis=0)
